# Initial kernel scaffold; baseline (speedup 1.0000x reference)
import jax
import jax.numpy as jnp
from jax.experimental import pallas as pl


def kernel(z, W):
    raise NotImplementedError("write your pallas kernel here")


# speedup vs baseline: 1.3198x; 1.3198x over previous
import functools

import jax
import jax.numpy as jnp
from jax import lax
from jax.experimental import pallas as pl
from jax.experimental.pallas import tpu as pltpu
from jax.experimental.pallas import tpu_sc as plsc

N = 16384
K = 1024
D = 256
TN = 512
GRID = N // TN
COMMIT = 0.25

NW = 32
ROWS_PER_W = N // NW
CHUNK = 128
NCHUNK = ROWS_PER_W // CHUNK


def _dist_body(z_ref, zsq_ref, w_ref, wsq_ref, idx_ref, loss_ref):
    i = pl.program_id(0)
    s = lax.dot_general(z_ref[...], w_ref[...],
                        (((1,), (1,)), ((), ())),
                        preferred_element_type=jnp.float32)
    d = (zsq_ref[...] + wsq_ref[...]) - 2.0 * s
    m = jnp.min(d, axis=1, keepdims=True)
    iota = lax.broadcasted_iota(jnp.int32, (TN, K), 1)
    idx_ref[...] = jnp.min(jnp.where(d == m, iota, K), axis=1, keepdims=True)

    @pl.when(i == 0)
    def _():
        loss_ref[0, 0] = 0.0

    loss_ref[0, 0] += jnp.sum(m)

    @pl.when(i == GRID - 1)
    def _():
        loss_ref[0, 0] = loss_ref[0, 0] * ((1.0 + COMMIT) / (N * D))


def _dist(z, z_sq, W, w_sq):
    return pl.pallas_call(
        _dist_body,
        grid=(GRID,),
        in_specs=[
            pl.BlockSpec((TN, D), lambda i: (i, 0)),
            pl.BlockSpec((TN, 1), lambda i: (i, 0)),
            pl.BlockSpec((K, D), lambda i: (0, 0)),
            pl.BlockSpec((1, K), lambda i: (0, 0)),
        ],
        out_specs=[
            pl.BlockSpec((TN, 1), lambda i: (i, 0)),
            pl.BlockSpec((1, 1), lambda i: (0, 0), memory_space=pltpu.SMEM),
        ],
        out_shape=[
            jax.ShapeDtypeStruct((N, 1), jnp.int32),
            jax.ShapeDtypeStruct((1, 1), jnp.float32),
        ],
    )(z, z_sq, W, w_sq)


@functools.cache
def _make_gather():
    @functools.partial(
        pl.kernel,
        mesh=plsc.VectorSubcoreMesh(core_axis_name="c", subcore_axis_name="s"),
        out_type=jax.ShapeDtypeStruct((N, D), jnp.float32),
        scratch_types=[
            pltpu.VMEM((CHUNK,), jnp.int32),
            pltpu.VMEM((CHUNK, D), jnp.float32),
            pltpu.SemaphoreType.DMA,
        ],
    )
    def _gather(w_hbm, idx_hbm, out_hbm, idx_v, rows_v, sem):
        wid = lax.axis_index("s") * 2 + lax.axis_index("c")
        base0 = wid * ROWS_PER_W
        for c in range(NCHUNK):
            base = base0 + c * CHUNK
            pltpu.sync_copy(idx_hbm.at[pl.ds(base, CHUNK)], idx_v)
            pltpu.async_copy(w_hbm.at[idx_v], rows_v, sem).wait()
            pltpu.sync_copy(rows_v, out_hbm.at[pl.ds(base, CHUNK)])

    return _gather


def kernel(z, W):
    z_sq = jnp.sum(z ** 2, axis=-1, keepdims=True)
    w_sq = jnp.sum(jnp.transpose(W) ** 2, axis=0, keepdims=True)
    idx2, loss = _dist(z, z_sq, W, w_sq)
    idx = idx2.reshape(N)
    z_q = _make_gather()(W, idx)
    return (z_q, loss[0, 0], idx)


# speedup vs baseline: 1.8093x; 1.3709x over previous
import functools

import jax
import jax.numpy as jnp
from jax import lax
from jax.experimental import pallas as pl
from jax.experimental.pallas import tpu as pltpu
from jax.experimental.pallas import tpu_sc as plsc

N = 16384
K = 1024
D = 256
TN = 1024
GRID = N // TN
COMMIT = 0.25

NW = 32
ROWS_PER_W = N // NW
CHUNK = 128
NCHUNK = ROWS_PER_W // CHUNK


def _dist_body(z_ref, w_ref, wsq_ref, idx_ref, loss_ref):
    i = pl.program_id(0)
    s = lax.dot_general(z_ref[...], w_ref[...],
                        (((1,), (1,)), ((), ())),
                        preferred_element_type=jnp.float32)
    zsq = jnp.sum(z_ref[...] ** 2, axis=1, keepdims=True)
    d = (zsq + wsq_ref[...]) - 2.0 * s
    m = jnp.min(d, axis=1, keepdims=True)
    iota = lax.broadcasted_iota(jnp.int32, (TN, K), 1).astype(jnp.float32)
    idxf = jnp.min(jnp.where(d == m, iota, float(K)), axis=1, keepdims=True)
    idx_ref[...] = idxf.astype(jnp.int32)

    @pl.when(i == 0)
    def _():
        loss_ref[0, 0] = 0.0

    loss_ref[0, 0] += jnp.sum(m)

    @pl.when(i == GRID - 1)
    def _():
        loss_ref[0, 0] = loss_ref[0, 0] * ((1.0 + COMMIT) / (N * D))


def _dist(z, W, w_sq):
    return pl.pallas_call(
        _dist_body,
        grid=(GRID,),
        in_specs=[
            pl.BlockSpec((TN, D), lambda i: (i, 0)),
            pl.BlockSpec((K, D), lambda i: (0, 0)),
            pl.BlockSpec((1, K), lambda i: (0, 0)),
        ],
        out_specs=[
            pl.BlockSpec((TN, 1), lambda i: (i, 0)),
            pl.BlockSpec((1, 1), lambda i: (0, 0), memory_space=pltpu.SMEM),
        ],
        out_shape=[
            jax.ShapeDtypeStruct((N, 1), jnp.int32),
            jax.ShapeDtypeStruct((1, 1), jnp.float32),
        ],
    )(z, W, w_sq)


@functools.cache
def _make_gather():
    @functools.partial(
        pl.kernel,
        mesh=plsc.VectorSubcoreMesh(core_axis_name="c", subcore_axis_name="s"),
        out_type=jax.ShapeDtypeStruct((N, D), jnp.float32),
        scratch_types=[
            pltpu.VMEM((CHUNK,), jnp.int32),
            pltpu.VMEM((CHUNK, D), jnp.float32),
            pltpu.SemaphoreType.DMA,
        ],
    )
    def _gather(w_hbm, idx_hbm, out_hbm, idx_v, rows_v, sem):
        wid = lax.axis_index("s") * 2 + lax.axis_index("c")
        base0 = wid * ROWS_PER_W
        for c in range(NCHUNK):
            base = base0 + c * CHUNK
            pltpu.sync_copy(idx_hbm.at[pl.ds(base, CHUNK)], idx_v)
            pltpu.async_copy(w_hbm.at[idx_v], rows_v, sem).wait()
            pltpu.sync_copy(rows_v, out_hbm.at[pl.ds(base, CHUNK)])

    return _gather


def kernel(z, W):
    w_sq = jnp.sum(jnp.transpose(W) ** 2, axis=0, keepdims=True)
    idx2, loss = _dist(z, W, w_sq)
    idx = idx2.reshape(N)
    z_q = _make_gather()(W, idx)
    return (z_q, loss[0, 0], idx)
